# baseline (speedup 1.0000x reference)
import functools

import jax
import jax.numpy as jnp
from jax import lax
from jax.experimental import pallas as pl
from jax.experimental.pallas import tpu as pltpu
from jax.experimental.pallas import tpu_sc as plsc

B, N, D, M = 16, 1024, 768, 300
MP = 304
HB = 8
LANES = 16


def _sc_gather_half(seg_flat, idx_flat, b0):
    mesh = plsc.VectorSubcoreMesh(core_axis_name="c", subcore_axis_name="s")

    @functools.partial(
        pl.kernel,
        mesh=mesh,
        out_type=jax.ShapeDtypeStruct((HB, M, D), jnp.float32),
        scratch_types=[
            pltpu.VMEM((80,), jnp.int32),
            pltpu.VMEM((LANES,), jnp.int32),
            pltpu.VMEM((80, D), jnp.float32),
            pltpu.SemaphoreType.DMA,
        ],
    )
    def k(seg_hbm, idx_hbm, out_hbm, idx_v, didx_v, rows_v, sem):
        wid = lax.axis_index("s") * 2 + lax.axis_index("c")
        bl = wid // 4
        q = wid % 4
        b = b0 + bl
        row_off = b * N
        iot = lax.iota(jnp.int32, 16)

        @pl.when(q < 3)
        def _():
            pltpu.sync_copy(idx_hbm.at[pl.ds(b * MP + q * 80, 80)], idx_v)
            for j in range(5):
                sl = pl.ds(j * LANES, LANES)
                idx_v[sl] = idx_v[sl] + row_off
            pltpu.async_copy(
                seg_hbm.at[idx_v], rows_v, sem).wait()
            pltpu.sync_copy(rows_v, out_hbm.at[bl, pl.ds(q * 80, 80), :])

        @pl.when(q == 3)
        def _():
            pltpu.sync_copy(
                idx_hbm.at[pl.ds(b * MP + 240, 64)], idx_v.at[pl.ds(0, 64)]
            )
            for j in range(4):
                sl = pl.ds(j * LANES, LANES)
                idx_v[sl] = idx_v[sl] + row_off
            didx_v[...] = jnp.minimum(288 + iot, M - 1)
            pltpu.async_copy(
                seg_hbm.at[idx_v.at[pl.ds(0, 64)]],
                rows_v.at[pl.ds(0, 64)], sem).wait()
            pltpu.sync_copy(
                rows_v.at[pl.ds(0, 56)], out_hbm.at[bl, pl.ds(240, 56), :]
            )
            pltpu.async_copy(
                rows_v.at[pl.ds(48, LANES)], out_hbm.at[bl].at[didx_v], sem
            ).wait()

    return k(seg_flat, idx_flat)


def kernel(seg_out, det_out, det_scores, det_indices):
    idx = det_indices.astype(jnp.int32)
    idx = jnp.pad(idx, ((0, 0), (0, MP - M)), mode="edge").reshape(B * MP)
    seg = seg_out.reshape(B * N, D)
    out_a = _sc_gather_half(seg, idx, 0)
    out_b = _sc_gather_half(seg, idx, HB)
    return jnp.concatenate([out_a, out_b], axis=0)


# speedup vs baseline: 1.2939x; 1.2939x over previous
import functools

import jax
import jax.numpy as jnp
from jax import lax
from jax.experimental import pallas as pl
from jax.experimental.pallas import tpu as pltpu
from jax.experimental.pallas import tpu_sc as plsc

B, N, D, M = 16, 1024, 768, 300
MP = 304
PW0 = 160
PW1 = 136
MT = 296
LANES = 16


def _sc_gather(seg_flat, idx_flat):
    mesh = plsc.VectorSubcoreMesh(core_axis_name="c", subcore_axis_name="s")

    @functools.partial(
        pl.kernel,
        mesh=mesh,
        out_type=jax.ShapeDtypeStruct((B, M, D), jnp.float32),
        scratch_types=[
            pltpu.VMEM((PW0,), jnp.int32),
            pltpu.VMEM((PW0, D), jnp.float32),
            pltpu.SemaphoreType.DMA,
        ],
    )
    def k(seg_hbm, idx_hbm, out_hbm, idx_v, rows_v, sem):
        wid = lax.axis_index("s") * 2 + lax.axis_index("c")
        b = wid // 2
        half = wid % 2
        row_off = b * N

        @pl.when(half == 0)
        def _():
            pltpu.sync_copy(idx_hbm.at[pl.ds(b * MP, PW0)], idx_v)
            for j in range(PW0 // LANES):
                sl = pl.ds(j * LANES, LANES)
                idx_v[sl] = idx_v[sl] + row_off
            for c in range(2):
                pltpu.async_copy(
                    seg_hbm.at[idx_v.at[pl.ds(c * 80, 80)]],
                    rows_v.at[pl.ds(c * 80, 80)],
                    sem,
                ).wait()
            pltpu.sync_copy(rows_v, out_hbm.at[b, pl.ds(0, PW0), :])

        @pl.when(half == 1)
        def _():
            pltpu.sync_copy(
                idx_hbm.at[pl.ds(b * MP + PW0, 144)], idx_v.at[pl.ds(0, 144)]
            )
            for j in range(144 // LANES):
                sl = pl.ds(j * LANES, LANES)
                idx_v[sl] = idx_v[sl] + row_off
            pltpu.async_copy(
                seg_hbm.at[idx_v.at[pl.ds(0, 80)]],
                rows_v.at[pl.ds(0, 80)],
                sem,
            ).wait()
            pltpu.async_copy(
                seg_hbm.at[idx_v.at[pl.ds(80, 64)]],
                rows_v.at[pl.ds(80, 64)],
                sem,
            ).wait()
            pltpu.sync_copy(
                rows_v.at[pl.ds(0, PW1)], out_hbm.at[b, pl.ds(PW0, PW1), :]
            )

    return k(seg_flat, idx_flat)


def kernel(seg_out, det_out, det_scores, det_indices):
    idx = det_indices.astype(jnp.int32)
    idx_padded = jnp.pad(idx, ((0, 0), (0, MP - M)))
    out = _sc_gather(seg_out.reshape(B * N, D), idx_padded.reshape(B * MP))
    tail = jnp.take_along_axis(seg_out, idx[:, MT:M, None], axis=1)
    return lax.dynamic_update_slice(out, tail, (0, MT, 0))
